# baseline (speedup 1.0000x reference)
import functools

import jax
import jax.numpy as jnp
from jax import lax
from jax.experimental import pallas as pl
from jax.experimental.pallas import tpu as pltpu
from jax.experimental.pallas import tpu_sc as plsc

D_MODEL = 2816
N_EXPERTS = 64
TOP_K = 8
RMS_EPS = 1e-06
_DSCALE = D_MODEL ** -0.5


def _scores_kernel(x_ref, w_ref, scale_ref, out_ref):
    x = x_ref[...]
    v = jnp.mean(x * x, axis=-1, keepdims=True)
    h = x * jax.lax.rsqrt(v + RMS_EPS)
    h = h * (scale_ref[...] * _DSCALE)
    out_ref[...] = jax.lax.dot_general(
        w_ref[...], h, (((1,), (1,)), ((), ())),
        preferred_element_type=jnp.float32)


@functools.partial(jax.jit, static_argnames=("block_t",))
def _tc_scores(x2d, W, scale, block_t):
    n_tok = x2d.shape[0]
    return pl.pallas_call(
        _scores_kernel,
        grid=(n_tok // block_t,),
        in_specs=[
            pl.BlockSpec((block_t, D_MODEL), lambda i: (i, 0)),
            pl.BlockSpec((N_EXPERTS, D_MODEL), lambda i: (0, 0)),
            pl.BlockSpec((1, D_MODEL), lambda i: (0, 0)),
        ],
        out_specs=pl.BlockSpec((N_EXPERTS, block_t), lambda i: (0, i)),
        out_shape=jax.ShapeDtypeStruct((N_EXPERTS, n_tok), jnp.float32),
    )(x2d, W, scale.reshape(1, D_MODEL))


def _make_sc_router(n_tok):
    info = plsc.get_sparse_core_info()
    NC, NS, L = info.num_cores, info.num_subcores, info.num_lanes
    NW = NC * NS
    t_per_w = n_tok // NW
    n_grp = t_per_w // L

    @functools.partial(
        pl.kernel,
        mesh=plsc.VectorSubcoreMesh(core_axis_name="c", subcore_axis_name="s"),
        out_type=jax.ShapeDtypeStruct((N_EXPERTS, n_tok), jnp.float32),
        scratch_types=[
            pltpu.VMEM((N_EXPERTS, t_per_w), jnp.float32),
            pltpu.VMEM((N_EXPERTS, t_per_w), jnp.float32),
            pltpu.VMEM((N_EXPERTS, L), jnp.float32),
            pltpu.SemaphoreType.DMA,
        ],
    )
    def sc_router(scores_hbm, pes_hbm, out_hbm, s_v, o_v, pes_v, sem):
        wid = lax.axis_index("s") * NC + lax.axis_index("c")
        base = wid * t_per_w
        pltpu.async_copy(scores_hbm.at[:, pl.ds(base, t_per_w)], s_v, sem).wait()
        pltpu.sync_copy(pes_hbm, pes_v)

        def group(g, carry):
            neg = jnp.full((L,), -jnp.inf, jnp.float32)
            zero_f = jnp.zeros((L,), jnp.float32)
            zero_i = jnp.zeros((L,), jnp.int32)
            one_i = jnp.full((L,), 1, jnp.int32)
            topk_i = jnp.full((L,), TOP_K, jnp.int32)
            one_f = jnp.full((L,), 1.0, jnp.float32)
            col = pl.ds(g * L, L)
            r = [neg] * TOP_K
            for e in range(N_EXPERTS):
                t = s_v[e, col]
                for j in range(TOP_K):
                    hi = jnp.maximum(r[j], t)
                    t = jnp.minimum(r[j], t)
                    r[j] = hi
            rowmax, t8 = r[0], r[TOP_K - 1]
            n_gt = zero_i
            for e in range(N_EXPERTS):
                n_gt = n_gt + jnp.where(s_v[e, col] > t8, one_i, zero_i)
            quota = topk_i - n_gt
            denom = zero_f
            cnt_eq = zero_i
            for e in range(N_EXPERTS):
                sv = s_v[e, col]
                eq = sv == t8
                sel = (sv > t8) | (eq & (cnt_eq < quota))
                cnt_eq = cnt_eq + jnp.where(eq, one_i, zero_i)
                ev = jnp.where(sel, jnp.exp(sv - rowmax), zero_f)
                denom = denom + ev
                o_v[e, col] = ev
            inv = one_f / denom
            for e in range(N_EXPERTS):
                o_v[e, col] = (o_v[e, col] * inv) * pes_v[e]
            return carry

        lax.fori_loop(0, n_grp, group, 0)
        pltpu.async_copy(o_v, out_hbm.at[:, pl.ds(base, t_per_w)], sem).wait()

    return sc_router


@jax.jit
def _hybrid(x2d, W, scale, pes):
    n_tok = x2d.shape[0]
    scores = _tc_scores(x2d, W, scale, block_t=2048)
    pes2 = jnp.asarray(
        jnp.broadcast_to(pes.reshape(N_EXPERTS, 1), (N_EXPERTS, 16)))
    return _make_sc_router(n_tok)(scores, pes2)


def kernel(x, W, scale, per_expert_scale):
    b, t, d = x.shape
    x2d = x.reshape(b * t, d)
    out_t = _hybrid(x2d, W, scale, per_expert_scale)
    return out_t.T.reshape(b, t, N_EXPERTS)


# speedup vs baseline: 1.6071x; 1.6071x over previous
import functools

import jax
import jax.numpy as jnp
from jax.experimental import pallas as pl
from jax.experimental.pallas import tpu as pltpu

D_MODEL = 2816
N_EXPERTS = 64
TOP_K = 8
RMS_EPS = 1e-06
_DSCALE = D_MODEL ** -0.5


def _router_kernel(x_ref, w_ref, scale_ref, pes_ref, out_ref):
    x = x_ref[...]
    v = jnp.mean(x * x, axis=-1, keepdims=True)
    h = x * jax.lax.rsqrt(v + RMS_EPS)
    h = h * (scale_ref[...] * _DSCALE)
    s = jax.lax.dot_general(
        w_ref[...], h, (((1,), (1,)), ((), ())),
        preferred_element_type=jnp.float32)

    iota = jax.lax.broadcasted_iota(jnp.int32, s.shape, 0)
    neg = jnp.float32(-jnp.inf)
    remaining = s
    sel = jnp.zeros(s.shape, dtype=jnp.bool_)
    rowmax = None
    for it in range(TOP_K):
        m = jnp.max(remaining, axis=0, keepdims=True)
        if it == 0:
            rowmax = m
        ismax = remaining == m
        amin = jnp.min(jnp.where(ismax, iota, N_EXPERTS), axis=0, keepdims=True)
        first = iota == amin
        sel = jnp.logical_or(sel, first)
        remaining = jnp.where(first, neg, remaining)

    e = jnp.where(sel, jnp.exp(s - rowmax), 0.0)
    denom = jnp.sum(e, axis=0, keepdims=True)
    out_ref[...] = (e / denom) * pes_ref[...][:, 0:1]


@functools.partial(jax.jit, static_argnames=("block_t",))
def _run(x2d, W, scale, per_expert_scale, block_t):
    n_tok = x2d.shape[0]
    grid = (n_tok // block_t,)
    pes2 = jnp.broadcast_to(
        per_expert_scale.reshape(N_EXPERTS, 1), (N_EXPERTS, 128))
    out_t = pl.pallas_call(
        _router_kernel,
        grid=grid,
        in_specs=[
            pl.BlockSpec((block_t, D_MODEL), lambda i: (i, 0)),
            pl.BlockSpec((N_EXPERTS, D_MODEL), lambda i: (0, 0)),
            pl.BlockSpec((1, D_MODEL), lambda i: (0, 0)),
            pl.BlockSpec((N_EXPERTS, 128), lambda i: (0, 0)),
        ],
        out_specs=pl.BlockSpec((N_EXPERTS, block_t), lambda i: (0, i)),
        out_shape=jax.ShapeDtypeStruct((N_EXPERTS, n_tok), jnp.float32),
        compiler_params=pltpu.CompilerParams(
            dimension_semantics=("parallel",),
        ),
    )(x2d, W, scale.reshape(1, D_MODEL), pes2)
    return out_t


def kernel(x, W, scale, per_expert_scale):
    b, t, d = x.shape
    x2d = x.reshape(b * t, d)
    out_t = _run(x2d, W, scale, per_expert_scale, block_t=2048)
    return out_t.T.reshape(b, t, N_EXPERTS)
